# Initial kernel scaffold; baseline (speedup 1.0000x reference)
import jax
import jax.numpy as jnp
from jax.experimental import pallas as pl


def kernel(gene_matrix, protein_matrix, adjacency_matrix, W1, b1, W2, b2, pe_W, pe_b, mn_W, mn_b, vr_W, vr_b, dec_W, dec_b, omega):
    raise NotImplementedError("write your pallas kernel here")


# speedup vs baseline: 1.8253x; 1.8253x over previous
import functools

import jax
import jax.numpy as jnp
from jax.experimental import pallas as pl
from jax.experimental.pallas import tpu as pltpu

_INTERPRET = False


def _lr(x):
    return jnp.where(x >= 0, x, 0.01 * x)


def _k1_body(gene_ref, w_ref, deg_ref, xw_ref, y_ref, dinv_ref):
    xw = jnp.dot(gene_ref[...], w_ref[...], preferred_element_type=jnp.float32)
    dinv = jax.lax.rsqrt(deg_ref[...])
    dinv_ref[...] = dinv
    drep = jnp.repeat(dinv, 64, axis=1)
    xw_ref[...] = xw
    y_ref[...] = xw * drep


def _k2_body(scat_ref, xw_ref, dinv_ref, b_ref, w2_ref, xw2_ref, y2_ref):
    dinv = dinv_ref[...]
    drep = jnp.repeat(dinv, 64, axis=1)
    h1 = _lr(drep * (scat_ref[...] + drep * xw_ref[...]) + b_ref[...])
    xw2 = jnp.dot(h1, w2_ref[...], preferred_element_type=jnp.float32)
    xw2_ref[...] = xw2
    y2_ref[...] = xw2 * jnp.repeat(dinv, 32, axis=1)


def _k3_body(scat_ref, xw_ref, dinv_ref, b_ref, prot_ref, pew_ref, peb_ref,
             mnw_ref, mnb_ref, vrw_ref, vrb_ref, decw_ref, decb_ref, om_ref,
             h2_ref, mu_ref, lv_ref, c0_ref, pr_ref):
    dinv = dinv_ref[...]
    drep = jnp.repeat(dinv, 32, axis=1)
    h2 = _lr(drep * (scat_ref[...] + drep * xw_ref[...]) + b_ref[...])
    h2_ref[...] = h2
    enc = _lr(jnp.dot(prot_ref[...], pew_ref[...],
                      preferred_element_type=jnp.float32) + peb_ref[...])
    mu = _lr(jnp.dot(enc, mnw_ref[...],
                     preferred_element_type=jnp.float32) + mnb_ref[...])
    lv = _lr(jnp.dot(enc, vrw_ref[...],
                     preferred_element_type=jnp.float32) + vrb_ref[...])
    mu_ref[...] = mu
    lv_ref[...] = lv
    w0 = om_ref[0, 0]
    w1 = om_ref[0, 1]
    gex = h2[:, 64:96]
    c0 = (w0 * gex + w1 * mu) / (w0 + w1)
    c0_ref[...] = c0
    pr_ref[...] = _lr(jnp.dot(c0, decw_ref[...],
                              preferred_element_type=jnp.float32) + decb_ref[...])


def _k4_body(a_ref, b_ref, o_ref):
    o_ref[...] = jax.lax.dot_general(
        a_ref[...], b_ref[...], (((1,), (1,)), ((), ())),
        preferred_element_type=jnp.float32)


def kernel(gene_matrix, protein_matrix, adjacency_matrix, W1, b1, W2, b2,
           pe_W, pe_b, mn_W, mn_b, vr_W, vr_b, dec_W, dec_b, omega):
    N, G = gene_matrix.shape
    P = protein_matrix.shape[1]
    L = W2.shape[1]
    F1, F2 = 2 * L, L
    S = 32 * N

    src, dst = jnp.nonzero(adjacency_matrix, size=S, fill_value=0)
    E = jnp.count_nonzero(adjacency_matrix)
    idx = jnp.arange(S)
    valid = (idx < E).astype(jnp.float32)
    mk = jax.random.key(42)
    if jax.config.jax_threefry_partitionable:
        u1 = jax.random.uniform(jax.random.fold_in(mk, 1), (S,))
        u2 = jax.random.uniform(jax.random.fold_in(mk, 2), (S,))
    else:
        Eu = E.astype(jnp.uint32)
        u1 = _unif_prefix(jax.random.key_data(jax.random.fold_in(mk, 1)), S, Eu)
        u2 = _unif_prefix(jax.random.key_data(jax.random.fold_in(mk, 2)), S, Eu)
    m1 = (u1 >= 0.4).astype(jnp.float32) * valid
    m2 = (u2 >= 0.5).astype(jnp.float32) * valid
    f1 = (jax.random.uniform(jax.random.fold_in(mk, 3), (G,)) >= 0.3).astype(jnp.float32)
    f2 = (jax.random.uniform(jax.random.fold_in(mk, 4), (G,)) >= 0.2).astype(jnp.float32)
    masks = jnp.stack([m1, m2, valid], 1)

    W1cat = jnp.concatenate([W1 * f1[:, None], W1 * f2[:, None], W1], axis=1)
    z = jnp.zeros((2 * L, L), jnp.float32)
    W2bd = jnp.block([[W2, z, z], [z, W2, z], [z, z, W2]])
    b1t = jnp.tile(b1, 3)[None, :]
    b2t = jnp.tile(b2, 3)[None, :]

    deg = jnp.zeros((N, 3), jnp.float32).at[dst].add(masks) + 1.0

    xw1, y1, dinv = pl.pallas_call(
        _k1_body,
        out_shape=(jax.ShapeDtypeStruct((N, 3 * F1), jnp.float32),
                   jax.ShapeDtypeStruct((N, 3 * F1), jnp.float32),
                   jax.ShapeDtypeStruct((N, 3), jnp.float32)),
        interpret=_INTERPRET,
    )(gene_matrix, W1cat, deg)

    scat1 = jnp.zeros((N, 3 * F1), jnp.float32).at[dst].add(
        y1[src] * jnp.repeat(masks, F1, axis=1))

    xw2, y2 = pl.pallas_call(
        _k2_body,
        out_shape=(jax.ShapeDtypeStruct((N, 3 * F2), jnp.float32),
                   jax.ShapeDtypeStruct((N, 3 * F2), jnp.float32)),
        interpret=_INTERPRET,
    )(scat1, xw1, dinv, b1t, W2bd)

    scat2 = jnp.zeros((N, 3 * F2), jnp.float32).at[dst].add(
        y2[src] * jnp.repeat(masks, F2, axis=1))

    h2, mu, logvar, c0, pex_recons = pl.pallas_call(
        _k3_body,
        out_shape=(jax.ShapeDtypeStruct((N, 3 * F2), jnp.float32),
                   jax.ShapeDtypeStruct((N, L), jnp.float32),
                   jax.ShapeDtypeStruct((N, L), jnp.float32),
                   jax.ShapeDtypeStruct((N, L), jnp.float32),
                   jax.ShapeDtypeStruct((N, P), jnp.float32)),
        interpret=_INTERPRET,
    )(scat2, xw2, dinv, b2t, protein_matrix, pe_W, pe_b[None, :],
      mn_W, mn_b[None, :], vr_W, vr_b[None, :], dec_W, dec_b[None, :],
      omega[None, :])

    BM = 600
    adj_recon = pl.pallas_call(
        _k4_body,
        grid=(N // BM,),
        in_specs=[pl.BlockSpec((BM, L), lambda i: (i, 0)),
                  pl.BlockSpec((N, L), lambda i: (0, 0))],
        out_specs=pl.BlockSpec((BM, N), lambda i: (i, 0)),
        out_shape=jax.ShapeDtypeStruct((N, N), jnp.float32),
        interpret=_INTERPRET,
    )(c0, c0)

    z1, z2, gex_z = h2[:, :L], h2[:, L:2 * L], h2[:, 2 * L:]
    return (adj_recon, pex_recons, z1, z2, gex_z, mu, mu, logvar, c0, c0, omega)


def _tf2x32(k0, k1, x0, x1):
    def rotl(x, d):
        return (x << jnp.uint32(d)) | (x >> jnp.uint32(32 - d))
    ks = (k0, k1, k0 ^ k1 ^ jnp.uint32(0x1BD11BDA))
    x0 = x0 + ks[0]
    x1 = x1 + ks[1]
    rotations = ((13, 15, 26, 6), (17, 29, 16, 24))
    for i in range(1, 6):
        for r in rotations[(i - 1) % 2]:
            x0 = x0 + x1
            x1 = rotl(x1, r)
            x1 = x0 ^ x1
        x0 = x0 + ks[i % 3]
        x1 = x1 + ks[(i + 1) % 3] + jnp.uint32(i)
    return x0, x1


def _unif_prefix(kd, S, e):
    k0 = kd[0]
    k1 = kd[1]
    idx = jnp.arange(S, dtype=jnp.uint32)
    half = (e + jnp.uint32(1)) // jnp.uint32(2)
    c1a = jnp.where(half + idx < e, half + idx, jnp.uint32(0))
    a0, _ = _tf2x32(k0, k1, idx, c1a)
    c0b = jnp.where(idx >= half, idx - half, jnp.uint32(0))
    _, b1 = _tf2x32(k0, k1, c0b, idx)
    bits = jnp.where(idx < half, a0, b1)
    f = jax.lax.bitcast_convert_type(
        (bits >> jnp.uint32(9)) | jnp.uint32(0x3F800000), jnp.float32)
    return jnp.maximum(jnp.float32(0.0), f - jnp.float32(1.0))
